# baseline (speedup 1.0000x reference)
import functools
import jax
import jax.numpy as jnp
from jax import lax
from jax.experimental import pallas as pl
from jax.experimental.pallas import tpu as pltpu
from jax.experimental.pallas import tpu_sc as plsc

_N = 10000
_D = 128
_E = 320000

_NC = 2
_NS = 16
_NW = _NC * _NS
_K = 96
_C = 120
_G = 24
_NG = _C // _G
_NBUF = 3
_EP = _NW * _C * _K
_NPAD = 10112
_RPT = _NPAD // _NS

_RB = 1264
_NB = _NPAD // _RB


def _sc_segment_sum():
    mesh = plsc.VectorSubcoreMesh(core_axis_name="c", subcore_axis_name="s")

    @functools.partial(
        pl.kernel,
        out_type=(
            jax.ShapeDtypeStruct((_NC, _NPAD, _D), jnp.float32),
            jax.ShapeDtypeStruct((_NC, _NPAD), jnp.float32),
        ),
        mesh=mesh,
        scratch_types=[
            pltpu.VMEM_SHARED((_NPAD, _D), jnp.float32),
            pltpu.VMEM_SHARED((_NPAD,), jnp.float32),
            pltpu.VMEM((2, _G, _K), jnp.int32),
            pltpu.VMEM((2, _G, _K), jnp.int32),
            [pltpu.VMEM((_K, _D), jnp.float32)] * _NBUF,
            pltpu.VMEM((_K,), jnp.float32),
            [pltpu.SemaphoreType.DMA] * _NBUF,
            [pltpu.SemaphoreType.DMA] * _NBUF,
            pltpu.SemaphoreType.DMA,
            pltpu.SemaphoreType.DMA,
        ],
    )
    def k(table, srcm, dstm, z2, z1, agg_out, cnt_out,
          agg_sh, cnt_sh, src_r, dst_r, rows_l, ones_v,
          gsem_l, ssem_l, isem, csem):
        c = lax.axis_index("c")
        s = lax.axis_index("s")
        wid = c * _NS + s

        pltpu.sync_copy(z2.at[pl.ds(s * _RPT, _RPT)],
                        agg_sh.at[pl.ds(s * _RPT, _RPT)])

        @pl.when(s == 0)
        def _():
            pltpu.sync_copy(z1, cnt_sh)

        for i in range(_K // 16):
            ones_v[pl.ds(i * 16, 16)] = jnp.full((16,), 1.0, jnp.float32)

        base = wid * _C
        bufs = tuple(zip(rows_l, gsem_l, ssem_l))

        def wait_gather(half, row, b):
            pltpu.make_async_copy(table.at[src_r.at[half, row]], bufs[b][0],
                                  bufs[b][1]).wait()

        def issue_gather(half, row, b):
            pltpu.async_copy(table.at[src_r.at[half, row]], bufs[b][0],
                             bufs[b][1])

        def issue_scatter(half, row, b):
            rows, _, ssem = bufs[b]
            pltpu.async_copy(rows, agg_sh.at[dst_r.at[half, row]], ssem,
                             add=True)
            pltpu.async_copy(ones_v, cnt_sh.at[dst_r.at[half, row]], csem,
                             add=True)

        def wait_scatter(b):
            rows, _, ssem = bufs[b]
            pltpu.make_async_copy(rows, agg_sh.at[dst_r.at[0, 0]],
                                  ssem).wait()

        def wait_refill(q):
            pltpu.make_async_copy(srcm.at[pl.ds(base, _G)], src_r.at[q],
                                  isem).wait()
            pltpu.make_async_copy(dstm.at[pl.ds(base, _G)], dst_r.at[q],
                                  isem).wait()

        def issue_refill(g, half):
            nbase = base + g * _G
            pltpu.async_copy(srcm.at[pl.ds(nbase, _G)], src_r.at[half], isem)
            pltpu.async_copy(dstm.at[pl.ds(nbase, _G)], dst_r.at[half], isem)

        pltpu.sync_copy(srcm.at[pl.ds(base, _G)], src_r.at[0])
        pltpu.sync_copy(dstm.at[pl.ds(base, _G)], dst_r.at[0])

        plsc.subcore_barrier()

        for b in range(2):
            issue_gather(0, b, b)
        issue_refill(1, 1)

        def drain_cnt():
            for _ in range(_G):
                pltpu.make_async_copy(ones_v, cnt_sh.at[dst_r.at[0, 0]],
                                      csem).wait()

        for jg in range(_G):
            b = jg % _NBUF
            wait_gather(0, jg, b)
            issue_scatter(0, jg, b)
            if jg >= 1:
                wait_scatter((jg - 1) % _NBUF)
            if jg == _G - 2:
                wait_refill(1)
            if jg < _G - 2:
                issue_gather(0, jg + 2, (jg + 2) % _NBUF)
            else:
                issue_gather(1, jg + 2 - _G, (jg + 2) % _NBUF)
        drain_cnt()

        def group(g, _):
            p = lax.rem(g, 2)
            q = 1 - p
            for jg in range(_G):
                b = jg % _NBUF
                wait_gather(p, jg, b)
                issue_scatter(p, jg, b)
                wait_scatter((jg - 1) % _NBUF)
                if jg == 2:
                    issue_refill(g + 1, q)
                if jg == _G - 2:
                    wait_refill(q)
                if jg < _G - 2:
                    issue_gather(p, jg + 2, (jg + 2) % _NBUF)
                else:
                    issue_gather(q, jg + 2 - _G, (jg + 2) % _NBUF)
            drain_cnt()
            return ()

        lax.fori_loop(1, _NG - 1, group, ())

        p = (_NG - 1) % 2
        for jg in range(_G):
            b = jg % _NBUF
            wait_gather(p, jg, b)
            issue_scatter(p, jg, b)
            wait_scatter((jg - 1) % _NBUF)
            if jg < _G - 2:
                issue_gather(p, jg + 2, (jg + 2) % _NBUF)
        wait_scatter((_G - 1) % _NBUF)
        drain_cnt()

        plsc.subcore_barrier()

        pltpu.sync_copy(agg_sh.at[pl.ds(s * _RPT, _RPT)],
                        agg_out.at[c, pl.ds(s * _RPT, _RPT)])

        @pl.when(s == 0)
        def _():
            pltpu.sync_copy(cnt_sh, cnt_out.at[c])

    return k


def _tc_dense(relu):
    def body(agg_ref, cnt_ref, x_ref, wl_ref, wr_ref, b_ref, out_ref):
        cnt = cnt_ref[0] + cnt_ref[1]
        mean = (agg_ref[0] + agg_ref[1]) / jnp.clip(cnt, 1.0, None)
        out = (jnp.dot(mean, wl_ref[...], preferred_element_type=jnp.float32)
               + jnp.dot(x_ref[...], wr_ref[...],
                         preferred_element_type=jnp.float32)
               + b_ref[...])
        if relu:
            out = jnp.maximum(out, 0.0)
        out_ref[...] = out

    return pl.pallas_call(
        body,
        grid=(_NB,),
        in_specs=[
            pl.BlockSpec((_NC, _RB, _D), lambda i: (0, i, 0)),
            pl.BlockSpec((_NC, _RB, 1), lambda i: (0, i, 0)),
            pl.BlockSpec((_RB, _D), lambda i: (i, 0)),
            pl.BlockSpec((_D, _D), lambda i: (0, 0)),
            pl.BlockSpec((_D, _D), lambda i: (0, 0)),
            pl.BlockSpec((1, _D), lambda i: (0, 0)),
        ],
        out_specs=pl.BlockSpec((_RB, _D), lambda i: (i, 0)),
        out_shape=jax.ShapeDtypeStruct((_NPAD, _D), jnp.float32),
    )


_seg_cnt = _sc_segment_sum()
_dense_relu = _tc_dense(True)
_dense_lin = _tc_dense(False)


def kernel(x, edge_index, W1l, W1r, b1, W2l, W2r, b2):
    x_pad = jnp.zeros((_NPAD, _D), jnp.float32).at[:_N].set(x)
    pad = _N + (jnp.arange(_EP - _E, dtype=jnp.int32) % (_NPAD - _N))
    srcm = jnp.concatenate([edge_index[0], pad]).reshape(_NW * _C, _K)
    dstm = jnp.concatenate([edge_index[1], pad]).reshape(_NW * _C, _K)
    z2 = jnp.zeros((_NPAD, _D), jnp.float32)
    z1 = jnp.zeros((_NPAD,), jnp.float32)

    agg1, cnt = _seg_cnt(x_pad, srcm, dstm, z2, z1)
    cnt3 = cnt.reshape(_NC, _NPAD, 1)
    h = _dense_relu(agg1, cnt3, x_pad, W1l.T, W1r.T, b1.reshape(1, _D))

    agg2, _ = _seg_cnt(h, srcm, dstm, z2, z1)
    out = _dense_lin(agg2, cnt3, h, W2l.T, W2r.T, b2.reshape(1, _D))
    return out[:_N]


# speedup vs baseline: 1.0872x; 1.0872x over previous
import functools
import jax
import jax.numpy as jnp
from jax import lax
from jax.experimental import pallas as pl
from jax.experimental.pallas import tpu as pltpu
from jax.experimental.pallas import tpu_sc as plsc

_N = 10000
_D = 128
_E = 320000

_NC = 2
_NS = 16
_NW = _NC * _NS
_K = 128
_C = 80
_G = 8
_NG = _C // _G
_NBUF = 2
_EP = _NW * _C * _K
_NPAD = 10240
_RPT = _NPAD // _NS

_RB = 1280
_NB = _NPAD // _RB


def _sc_segment_sum():
    mesh = plsc.VectorSubcoreMesh(core_axis_name="c", subcore_axis_name="s")

    @functools.partial(
        pl.kernel,
        out_type=(
            jax.ShapeDtypeStruct((_NC, _NPAD, _D), jnp.float32),
            jax.ShapeDtypeStruct((_NC, _NPAD), jnp.float32),
        ),
        mesh=mesh,
        scratch_types=[
            pltpu.VMEM_SHARED((_NPAD, _D), jnp.float32),
            pltpu.VMEM_SHARED((_NPAD,), jnp.float32),
            pltpu.VMEM((2, _G, _K), jnp.int32),
            pltpu.VMEM((2, _G, _K), jnp.int32),
            [pltpu.VMEM((_K, _D), jnp.float32)] * _NBUF,
            pltpu.VMEM((_K,), jnp.float32),
            [pltpu.SemaphoreType.DMA] * _NBUF,
            pltpu.SemaphoreType.DMA,
            pltpu.SemaphoreType.DMA,
            pltpu.SemaphoreType.DMA,
        ],
    )
    def k(table, srcm, dstm, z2, z1, agg_out, cnt_out,
          agg_sh, cnt_sh, src_r, dst_r, rows_l, ones_v,
          gsem_l, isem, csem, zsem):
        c = lax.axis_index("c")
        s = lax.axis_index("s")
        wid = c * _NS + s

        pltpu.async_copy(z2.at[pl.ds(s * _RPT, _RPT)],
                         agg_sh.at[pl.ds(s * _RPT, _RPT)], zsem)

        @pl.when(s == 0)
        def _():
            pltpu.async_copy(z1, cnt_sh, zsem)

        for i in range(_K // 16):
            ones_v[pl.ds(i * 16, 16)] = jnp.full((16,), 1.0, jnp.float32)

        base = wid * _C
        bufs = tuple(zip(rows_l, gsem_l))

        def wait_gather(half, row, b):
            pltpu.make_async_copy(table.at[src_r.at[half, row]], bufs[b][0],
                                  bufs[b][1]).wait()

        def issue_gather(half, row, b):
            pltpu.async_copy(table.at[src_r.at[half, row]], bufs[b][0],
                             bufs[b][1])

        def issue_scatter(half, row, b):
            rows = bufs[b][0]
            pltpu.sync_copy(rows, agg_sh.at[dst_r.at[half, row]], add=True)
            pltpu.async_copy(ones_v, cnt_sh.at[dst_r.at[half, row]], csem,
                             add=True)

        def wait_refill(q):
            pltpu.make_async_copy(srcm.at[pl.ds(base, _G)], src_r.at[q],
                                  isem).wait()
            pltpu.make_async_copy(dstm.at[pl.ds(base, _G)], dst_r.at[q],
                                  isem).wait()

        def issue_refill(g, half):
            nbase = base + g * _G
            pltpu.async_copy(srcm.at[pl.ds(nbase, _G)], src_r.at[half], isem)
            pltpu.async_copy(dstm.at[pl.ds(nbase, _G)], dst_r.at[half], isem)

        pltpu.sync_copy(srcm.at[pl.ds(base, _G)], src_r.at[0])
        pltpu.sync_copy(dstm.at[pl.ds(base, _G)], dst_r.at[0])

        for b in range(2):
            issue_gather(0, b, b)
        issue_refill(1, 1)

        pltpu.make_async_copy(z2.at[pl.ds(s * _RPT, _RPT)],
                              agg_sh.at[pl.ds(s * _RPT, _RPT)], zsem).wait()

        @pl.when(s == 0)
        def _():
            pltpu.make_async_copy(z1, cnt_sh, zsem).wait()

        plsc.subcore_barrier()

        def drain_cnt():
            for _ in range(_G):
                pltpu.make_async_copy(ones_v, cnt_sh.at[dst_r.at[0, 0]],
                                      csem).wait()

        for jg in range(_G):
            b = jg % _NBUF
            wait_gather(0, jg, b)
            issue_scatter(0, jg, b)
            if jg == _G - 2:
                wait_refill(1)
            if jg < _G - 2:
                issue_gather(0, jg + 2, b)
            else:
                issue_gather(1, jg + 2 - _G, b)
        drain_cnt()

        def group(g, _):
            p = lax.rem(g, 2)
            q = 1 - p
            for jg in range(_G):
                b = jg % _NBUF
                wait_gather(p, jg, b)
                issue_scatter(p, jg, b)
                if jg == 2:
                    issue_refill(g + 1, q)
                if jg == _G - 2:
                    wait_refill(q)
                if jg < _G - 2:
                    issue_gather(p, jg + 2, b)
                else:
                    issue_gather(q, jg + 2 - _G, b)
            drain_cnt()
            return ()

        lax.fori_loop(1, _NG - 1, group, ())

        p = (_NG - 1) % 2
        for jg in range(_G):
            b = jg % _NBUF
            wait_gather(p, jg, b)
            issue_scatter(p, jg, b)
            if jg < _G - 2:
                issue_gather(p, jg + 2, b)
        drain_cnt()

        plsc.subcore_barrier()

        pltpu.sync_copy(agg_sh.at[pl.ds(s * _RPT, _RPT)],
                        agg_out.at[c, pl.ds(s * _RPT, _RPT)])

        @pl.when(s == 0)
        def _():
            pltpu.sync_copy(cnt_sh, cnt_out.at[c])

    return k


def _tc_dense(relu):
    def body(agg_ref, cnt_ref, x_ref, wl_ref, wr_ref, b_ref, out_ref):
        cnt = cnt_ref[0] + cnt_ref[1]
        mean = (agg_ref[0] + agg_ref[1]) / jnp.clip(cnt, 1.0, None)
        out = (jnp.dot(mean, wl_ref[...], preferred_element_type=jnp.float32)
               + jnp.dot(x_ref[...], wr_ref[...],
                         preferred_element_type=jnp.float32)
               + b_ref[...])
        if relu:
            out = jnp.maximum(out, 0.0)
        out_ref[...] = out

    return pl.pallas_call(
        body,
        grid=(_NB,),
        in_specs=[
            pl.BlockSpec((_NC, _RB, _D), lambda i: (0, i, 0)),
            pl.BlockSpec((_NC, _RB, 1), lambda i: (0, i, 0)),
            pl.BlockSpec((_RB, _D), lambda i: (i, 0)),
            pl.BlockSpec((_D, _D), lambda i: (0, 0)),
            pl.BlockSpec((_D, _D), lambda i: (0, 0)),
            pl.BlockSpec((1, _D), lambda i: (0, 0)),
        ],
        out_specs=pl.BlockSpec((_RB, _D), lambda i: (i, 0)),
        out_shape=jax.ShapeDtypeStruct((_NPAD, _D), jnp.float32),
    )


_seg_cnt = _sc_segment_sum()
_dense_relu = _tc_dense(True)
_dense_lin = _tc_dense(False)


def kernel(x, edge_index, W1l, W1r, b1, W2l, W2r, b2):
    x_pad = jnp.zeros((_NPAD, _D), jnp.float32).at[:_N].set(x)
    pad = _N + (jnp.arange(_EP - _E, dtype=jnp.int32) % (_NPAD - _N))
    srcm = jnp.concatenate([edge_index[0], pad]).reshape(_NW * _C, _K)
    dstm = jnp.concatenate([edge_index[1], pad]).reshape(_NW * _C, _K)
    z2 = jnp.zeros((_NPAD, _D), jnp.float32)
    z1 = jnp.zeros((_NPAD,), jnp.float32)

    agg1, cnt = _seg_cnt(x_pad, srcm, dstm, z2, z1)
    cnt3 = cnt.reshape(_NC, _NPAD, 1)
    h = _dense_relu(agg1, cnt3, x_pad, W1l.T, W1r.T, b1.reshape(1, _D))

    agg2, _ = _seg_cnt(h, srcm, dstm, z2, z1)
    out = _dense_lin(agg2, cnt3, h, W2l.T, W2r.T, b2.reshape(1, _D))
    return out[:_N]
